# baseline (speedup 1.0000x reference)
import functools

import jax
import jax.numpy as jnp
from jax import lax
from jax.experimental import pallas as pl
from jax.experimental.pallas import tpu as pltpu
from jax.experimental.pallas import tpu_sc as plsc

_B, _N, _C, _DFF, _K = 4, 8192, 768, 3072, 1024
_NC, _NS, _L = 2, 16, 16
_NW = _NC * _NS
_RPW = (_B * _K) // _NW
_SEARCH_ITERS = 31


def _sc_mesh():
    return plsc.VectorSubcoreMesh(
        core_axis_name="c", subcore_axis_name="s",
        num_cores=_NC, num_subcores=_NS)


_SC_PARAMS = pltpu.CompilerParams(needs_layout_passes=False)


def _wid():
    return lax.axis_index("s") * _NC + lax.axis_index("c")


def _prep_body(scores_ref, thr_ref):
    bits = lax.bitcast_convert_type(scores_ref[...], jnp.int32)

    def step(_, carry):
        lo, hi = carry
        mid = lo + (hi - lo) // 2
        cnt = jnp.sum((bits >= mid).astype(jnp.int32), axis=1, keepdims=True)
        ok = cnt >= _K
        return jnp.where(ok, mid, lo), jnp.where(ok, hi, mid)

    lo0 = jnp.zeros((_B, 1), jnp.int32)
    hi0 = jnp.full((_B, 1), 0x7F800000, jnp.int32)
    lo, _ = lax.fori_loop(0, _SEARCH_ITERS, step, (lo0, hi0))
    thr = lax.bitcast_convert_type(lo, jnp.float32)
    thr_ref[...] = jnp.broadcast_to(thr, (_B, _L))


def _prep_call(scores, *, interpret=False):
    return pl.pallas_call(
        _prep_body,
        out_shape=jax.ShapeDtypeStruct((_B, _L), jnp.float32),
        interpret=interpret,
    )(scores)


def _make_compact_gather(interpret=False):
    @functools.partial(
        pl.kernel,
        out_type=(jax.ShapeDtypeStruct((_B * _K,), jnp.int32),
                  jax.ShapeDtypeStruct((_B * _K, _C), jnp.float32)),
        mesh=_sc_mesh(),
        scratch_types=[
            pltpu.VMEM((_N,), jnp.float32),
            pltpu.VMEM((_L,), jnp.float32),
            pltpu.VMEM((2 * _K + _L,), jnp.int32),
            pltpu.VMEM((_K + _L,), jnp.int32),
            pltpu.VMEM((_RPW,), jnp.int32),
            pltpu.VMEM((_RPW, _C), jnp.float32),
            pltpu.VMEM_SHARED((2 * _K,), jnp.int32),
            pltpu.SemaphoreType.DMA,
        ],
        compiler_params=_SC_PARAMS,
        interpret=interpret,
    )
    def compact_gather(scores_hbm, thr_hbm, xflat_hbm, fidx_hbm, xf_hbm,
                       sc_v, thr_v, idxbuf, eqbuf, idx_v, rows_v,
                       shared_idx, sem):
        cid = lax.axis_index("c")
        sid = lax.axis_index("s")
        w = cid * _NS + sid

        @pl.when(jnp.logical_or(sid == 0, sid == 8))
        def _():
            b = cid * 2 + sid // 8
            pltpu.sync_copy(scores_hbm.at[b], sc_v)
            pltpu.sync_copy(thr_hbm.at[b], thr_v)
            thr_vec = thr_v[...]
            iota = lax.iota(jnp.int32, _L)
            base = b * _N

            trash_gt = 2 * _K + _L - 1
            trash_eq = _K

            def step(i, carry):
                off_gt, off_eq = carry
                v = sc_v[pl.ds(i * _L, _L)]
                idxv = iota + (base + i * _L)
                mgt = v > thr_vec
                inc_gt = mgt.astype(jnp.int32)
                cs_gt = plsc.cumsum(inc_gt)
                pos_gt = jnp.where(mgt, off_gt + cs_gt - 1, trash_gt)
                plsc.store_scatter(idxbuf, [pos_gt], idxv)
                ngt = jnp.sum(inc_gt, axis=0)
                meq = v == thr_vec
                inc_eq = meq.astype(jnp.int32)
                cs_eq = plsc.cumsum(inc_eq)
                pos_raw = off_eq + cs_eq - 1
                valid = meq & (pos_raw < _K)
                pos_eq = jnp.where(valid, pos_raw, trash_eq)
                plsc.store_scatter(eqbuf, [pos_eq], idxv)
                neq = jnp.sum(inc_eq, axis=0)
                return off_gt + ngt, off_eq + neq

            off_gt, _unused = lax.fori_loop(
                0, _N // _L, step, (jnp.int32(0), jnp.int32(0)))

            def app(j, carry):
                idxbuf[pl.ds(off_gt + j * _L, _L)] = eqbuf[pl.ds(j * _L, _L)]
                return carry

            lax.fori_loop(0, _K // _L, app, jnp.int32(0))
            pltpu.sync_copy(idxbuf.at[pl.ds(0, _K)],
                            shared_idx.at[pl.ds((sid // 8) * _K, _K)])

        plsc.subcore_barrier()

        pltpu.sync_copy(shared_idx.at[pl.ds(sid * _RPW, _RPW)], idx_v)
        pltpu.async_copy(xflat_hbm.at[idx_v], rows_v, sem).wait()
        base_out = w * _RPW
        pltpu.sync_copy(rows_v, xf_hbm.at[pl.ds(base_out, _RPW)])

        @pl.when(jnp.logical_or(sid == 0, sid == 8))
        def _():
            b = cid * 2 + sid // 8
            pltpu.sync_copy(idxbuf.at[pl.ds(0, _K)],
                            fidx_hbm.at[pl.ds(b * _K, _K)])

    return compact_gather


_GF = 16
_RB = 1024
_GC = 12
_TM = (_B * _K) // _GF
_NROWS = _B * _N
_RA = _NROWS - _GC * _RB
_RAB = 2048
_OFF = _RA // _RB


def _copy_a_body(src_ref, dst_ref):
    dst_ref[...] = src_ref[...]


def _copy_a_call(cache_flat, *, interpret=False):
    return pl.pallas_call(
        _copy_a_body,
        grid=(_RA // _RAB,),
        in_specs=[pl.BlockSpec((_RAB, _C), lambda i: (i, 0))],
        out_specs=pl.BlockSpec((_RAB, _C), lambda i: (i, 0)),
        out_shape=jax.ShapeDtypeStruct((_NROWS, _C), jnp.float32),
        interpret=interpret,
    )(cache_flat)


def _fused_body(alias_ref, cache_ref, xf_ref, w1_ref, b1_ref, w2_ref,
                b2_ref, copy_ref, out_ref):
    del alias_ref

    @pl.when(pl.program_id(0) < _GC)
    def _():
        copy_ref[...] = cache_ref[...]
    h = jnp.dot(xf_ref[...].astype(jnp.bfloat16),
                w1_ref[...].astype(jnp.bfloat16),
                preferred_element_type=jnp.float32)
    g = jax.nn.gelu((h + b1_ref[...]).astype(jnp.bfloat16))
    out = jnp.dot(g, w2_ref[...].astype(jnp.bfloat16),
                  preferred_element_type=jnp.float32)
    out_ref[...] = out + b2_ref[...]


def _fused_call(copied_a, cache_flat, xf, w1, b1, w2, b2, *, interpret=False):
    return pl.pallas_call(
        _fused_body,
        grid=(_GF,),
        in_specs=[
            pl.BlockSpec(memory_space=pl.ANY),
            pl.BlockSpec((_RB, _C),
                         lambda i: (jnp.minimum(i, _GC - 1) + _OFF, 0)),
            pl.BlockSpec((_TM, _C), lambda i: (i, 0)),
            pl.BlockSpec((_C, _DFF), lambda i: (0, 0)),
            pl.BlockSpec((1, _DFF), lambda i: (0, 0)),
            pl.BlockSpec((_DFF, _C), lambda i: (0, 0)),
            pl.BlockSpec((1, _C), lambda i: (0, 0)),
        ],
        out_specs=[
            pl.BlockSpec((_RB, _C),
                         lambda i: (jnp.minimum(i, _GC - 1) + _OFF, 0)),
            pl.BlockSpec((_TM, _C), lambda i: (i, 0)),
        ],
        out_shape=[
            jax.ShapeDtypeStruct((_NROWS, _C), jnp.float32),
            jax.ShapeDtypeStruct((_B * _K, _C), jnp.float32),
        ],
        input_output_aliases={0: 0},
        interpret=interpret,
    )(copied_a, cache_flat, xf, w1, b1, w2, b2)


def _make_scatter(interpret=False):
    @functools.partial(
        pl.kernel,
        out_type=(),
        mesh=_sc_mesh(),
        scratch_types=[
            pltpu.VMEM((_RPW,), jnp.int32),
            pltpu.VMEM((_RPW, _C), jnp.float32),
            pltpu.SemaphoreType.DMA,
        ],
        compiler_params=_SC_PARAMS,
        interpret=interpret,
    )
    def scatter(fidx_hbm, outf_hbm, dst_ref, idx_v, rows_v, sem):
        base = _wid() * _RPW
        pltpu.sync_copy(fidx_hbm.at[pl.ds(base, _RPW)], idx_v)
        pltpu.sync_copy(outf_hbm.at[pl.ds(base, _RPW)], rows_v)
        pltpu.async_copy(rows_v, dst_ref.at[idx_v], sem).wait()

    return scatter


def kernel(x, cache, scores, W1, b1, W2, b2):
    xflat = x.reshape(_B * _N, _C)
    cache_flat = cache.reshape(_NROWS, _C)
    copied_a = _copy_a_call(cache_flat)
    thr = _prep_call(scores)
    fidx, xf = _make_compact_gather()(scores, thr, xflat)
    copied, outf = _fused_call(copied_a, cache_flat, xf, W1,
                               b1.reshape(1, _DFF), W2, b2.reshape(1, _C))
    dst = jax.new_ref(copied)
    _make_scatter()(fidx, outf, dst)
    return dst[...].reshape(_B, _N, _C)


# speedup vs baseline: 1.0557x; 1.0557x over previous
import functools

import jax
import jax.numpy as jnp
from jax import lax
from jax.experimental import pallas as pl
from jax.experimental.pallas import tpu as pltpu
from jax.experimental.pallas import tpu_sc as plsc

_B, _N, _C, _DFF, _K = 4, 8192, 768, 3072, 1024
_NC, _NS, _L = 2, 16, 16
_NW = _NC * _NS
_RPW = (_B * _K) // _NW
_SEARCH_ITERS = 31


def _sc_mesh():
    return plsc.VectorSubcoreMesh(
        core_axis_name="c", subcore_axis_name="s",
        num_cores=_NC, num_subcores=_NS)


_SC_PARAMS = pltpu.CompilerParams(needs_layout_passes=False)


def _wid():
    return lax.axis_index("s") * _NC + lax.axis_index("c")


def _prep_body(scores_ref, thr_ref):
    bits = lax.bitcast_convert_type(scores_ref[...], jnp.int32)

    def step(_, carry):
        lo, hi = carry
        mid = lo + (hi - lo) // 2
        cnt = jnp.sum((bits >= mid).astype(jnp.int32), axis=1, keepdims=True)
        ok = cnt >= _K
        return jnp.where(ok, mid, lo), jnp.where(ok, hi, mid)

    lo0 = jnp.zeros((_B, 1), jnp.int32)
    hi0 = jnp.full((_B, 1), 0x7F800000, jnp.int32)
    lo, _ = lax.fori_loop(0, _SEARCH_ITERS, step, (lo0, hi0))
    thr = lax.bitcast_convert_type(lo, jnp.float32)
    thr_ref[...] = jnp.broadcast_to(thr, (_B, _L))


def _prep_call(scores, *, interpret=False):
    return pl.pallas_call(
        _prep_body,
        out_shape=jax.ShapeDtypeStruct((_B, _L), jnp.float32),
        interpret=interpret,
    )(scores)


def _make_compact_gather(interpret=False):
    @functools.partial(
        pl.kernel,
        out_type=(jax.ShapeDtypeStruct((_B * _K,), jnp.int32),
                  jax.ShapeDtypeStruct((_B * _K, _C), jnp.float32)),
        mesh=_sc_mesh(),
        scratch_types=[
            pltpu.VMEM((_N,), jnp.float32),
            pltpu.VMEM((_L,), jnp.float32),
            pltpu.VMEM((2 * _K + _L,), jnp.int32),
            pltpu.VMEM((_K + _L,), jnp.int32),
            pltpu.VMEM((_RPW,), jnp.int32),
            pltpu.VMEM((_RPW, _C), jnp.float32),
            pltpu.VMEM_SHARED((2 * _K,), jnp.int32),
            pltpu.SemaphoreType.DMA,
        ],
        compiler_params=_SC_PARAMS,
        interpret=interpret,
    )
    def compact_gather(scores_hbm, thr_hbm, xflat_hbm, fidx_hbm, xf_hbm,
                       sc_v, thr_v, idxbuf, eqbuf, idx_v, rows_v,
                       shared_idx, sem):
        cid = lax.axis_index("c")
        sid = lax.axis_index("s")
        w = cid * _NS + sid

        @pl.when(jnp.logical_or(sid == 0, sid == 8))
        def _():
            b = cid * 2 + sid // 8
            pltpu.sync_copy(scores_hbm.at[b], sc_v)
            pltpu.sync_copy(thr_hbm.at[b], thr_v)
            thr_vec = thr_v[...]
            iota = lax.iota(jnp.int32, _L)
            base = b * _N

            trash_gt = 2 * _K + _L - 1
            trash_eq = _K

            def step(i, carry):
                off_gt, off_eq = carry
                v = sc_v[pl.ds(i * _L, _L)]
                idxv = iota + (base + i * _L)
                mgt = v > thr_vec
                inc_gt = mgt.astype(jnp.int32)
                cs_gt = plsc.cumsum(inc_gt)
                pos_gt = jnp.where(mgt, off_gt + cs_gt - 1, trash_gt)
                plsc.store_scatter(idxbuf, [pos_gt], idxv)
                ngt = jnp.sum(inc_gt, axis=0)
                meq = v == thr_vec
                inc_eq = meq.astype(jnp.int32)
                cs_eq = plsc.cumsum(inc_eq)
                pos_raw = off_eq + cs_eq - 1
                valid = meq & (pos_raw < _K)
                pos_eq = jnp.where(valid, pos_raw, trash_eq)
                plsc.store_scatter(eqbuf, [pos_eq], idxv)
                neq = jnp.sum(inc_eq, axis=0)
                return off_gt + ngt, off_eq + neq

            off_gt, _unused = lax.fori_loop(
                0, _N // _L, step, (jnp.int32(0), jnp.int32(0)))

            def app(j, carry):
                idxbuf[pl.ds(off_gt + j * _L, _L)] = eqbuf[pl.ds(j * _L, _L)]
                return carry

            lax.fori_loop(0, _K // _L, app, jnp.int32(0))
            pltpu.sync_copy(idxbuf.at[pl.ds(0, _K)],
                            shared_idx.at[pl.ds((sid // 8) * _K, _K)])

        plsc.subcore_barrier()

        pltpu.sync_copy(shared_idx.at[pl.ds(sid * _RPW, _RPW)], idx_v)
        pltpu.async_copy(xflat_hbm.at[idx_v], rows_v, sem).wait()
        base_out = w * _RPW
        pltpu.sync_copy(rows_v, xf_hbm.at[pl.ds(base_out, _RPW)])

        @pl.when(jnp.logical_or(sid == 0, sid == 8))
        def _():
            b = cid * 2 + sid // 8
            pltpu.sync_copy(idxbuf.at[pl.ds(0, _K)],
                            fidx_hbm.at[pl.ds(b * _K, _K)])

    return compact_gather


_GF = 16
_RB = 1024
_GC = 16
_TM = (_B * _K) // _GF
_NROWS = _B * _N
_RA = _NROWS - _GC * _RB
_RAB = 2048
_OFF = _RA // _RB


def _copy_a_body(src_ref, dst_ref):
    dst_ref[...] = src_ref[...]


def _copy_a_call(cache_flat, *, interpret=False):
    return pl.pallas_call(
        _copy_a_body,
        grid=(_RA // _RAB,),
        in_specs=[pl.BlockSpec((_RAB, _C), lambda i: (i, 0))],
        out_specs=pl.BlockSpec((_RAB, _C), lambda i: (i, 0)),
        out_shape=jax.ShapeDtypeStruct((_NROWS, _C), jnp.float32),
        interpret=interpret,
    )(cache_flat)


def _fused_body(alias_ref, cache_ref, xf_ref, w1_ref, b1_ref, w2_ref,
                b2_ref, copy_ref, out_ref):
    del alias_ref

    @pl.when(pl.program_id(0) < _GC)
    def _():
        copy_ref[...] = cache_ref[...]
    h = jnp.dot(xf_ref[...].astype(jnp.bfloat16),
                w1_ref[...].astype(jnp.bfloat16),
                preferred_element_type=jnp.float32)
    g = jax.nn.gelu((h + b1_ref[...]).astype(jnp.bfloat16))
    out = jnp.dot(g, w2_ref[...].astype(jnp.bfloat16),
                  preferred_element_type=jnp.float32)
    out_ref[...] = out + b2_ref[...]


def _fused_call(copied_a, cache_flat, xf, w1, b1, w2, b2, *, interpret=False):
    return pl.pallas_call(
        _fused_body,
        grid=(_GF,),
        in_specs=[
            pl.BlockSpec(memory_space=pl.ANY),
            pl.BlockSpec((_RB, _C),
                         lambda i: (jnp.minimum(i, _GC - 1) + _OFF, 0)),
            pl.BlockSpec((_TM, _C), lambda i: (i, 0)),
            pl.BlockSpec((_C, _DFF), lambda i: (0, 0)),
            pl.BlockSpec((1, _DFF), lambda i: (0, 0)),
            pl.BlockSpec((_DFF, _C), lambda i: (0, 0)),
            pl.BlockSpec((1, _C), lambda i: (0, 0)),
        ],
        out_specs=[
            pl.BlockSpec((_RB, _C),
                         lambda i: (jnp.minimum(i, _GC - 1) + _OFF, 0)),
            pl.BlockSpec((_TM, _C), lambda i: (i, 0)),
        ],
        out_shape=[
            jax.ShapeDtypeStruct((_NROWS, _C), jnp.float32),
            jax.ShapeDtypeStruct((_B * _K, _C), jnp.float32),
        ],
        input_output_aliases={0: 0},
        interpret=interpret,
    )(copied_a, cache_flat, xf, w1, b1, w2, b2)


def _make_scatter(interpret=False):
    @functools.partial(
        pl.kernel,
        out_type=(),
        mesh=_sc_mesh(),
        scratch_types=[
            pltpu.VMEM((_RPW,), jnp.int32),
            pltpu.VMEM((_RPW, _C), jnp.float32),
            pltpu.SemaphoreType.DMA,
        ],
        compiler_params=_SC_PARAMS,
        interpret=interpret,
    )
    def scatter(fidx_hbm, outf_hbm, dst_ref, idx_v, rows_v, sem):
        base = _wid() * _RPW
        pltpu.sync_copy(fidx_hbm.at[pl.ds(base, _RPW)], idx_v)
        pltpu.sync_copy(outf_hbm.at[pl.ds(base, _RPW)], rows_v)
        pltpu.async_copy(rows_v, dst_ref.at[idx_v], sem).wait()

    return scatter


def kernel(x, cache, scores, W1, b1, W2, b2):
    xflat = x.reshape(_B * _N, _C)
    cache_flat = cache.reshape(_NROWS, _C)
    copied_a = _copy_a_call(cache_flat)
    thr = _prep_call(scores)
    fidx, xf = _make_compact_gather()(scores, thr, xflat)
    copied, outf = _fused_call(copied_a, cache_flat, xf, W1,
                               b1.reshape(1, _DFF), W2, b2.reshape(1, _C))
    dst = jax.new_ref(copied)
    _make_scatter()(fidx, outf, dst)
    return dst[...].reshape(_B, _N, _C)


# speedup vs baseline: 1.0929x; 1.0352x over previous
import functools

import jax
import jax.numpy as jnp
from jax import lax
from jax.experimental import pallas as pl
from jax.experimental.pallas import tpu as pltpu
from jax.experimental.pallas import tpu_sc as plsc

_B, _N, _C, _DFF, _K = 4, 8192, 768, 3072, 1024
_NC, _NS, _L = 2, 16, 16
_NW = _NC * _NS
_RPW = (_B * _K) // _NW
_SEARCH_ITERS = 31


def _sc_mesh():
    return plsc.VectorSubcoreMesh(
        core_axis_name="c", subcore_axis_name="s",
        num_cores=_NC, num_subcores=_NS)


_SC_PARAMS = pltpu.CompilerParams(needs_layout_passes=False)


def _wid():
    return lax.axis_index("s") * _NC + lax.axis_index("c")


def _prep_body(scores_ref, thr_ref):
    bits = lax.bitcast_convert_type(scores_ref[...], jnp.int32)

    def step(_, carry):
        lo, hi = carry
        mid = lo + (hi - lo) // 2
        cnt = jnp.sum((bits >= mid).astype(jnp.int32), axis=1, keepdims=True)
        ok = cnt >= _K
        return jnp.where(ok, mid, lo), jnp.where(ok, hi, mid)

    lo0 = jnp.zeros((_B, 1), jnp.int32)
    hi0 = jnp.full((_B, 1), 0x7F800000, jnp.int32)
    lo, _ = lax.fori_loop(0, _SEARCH_ITERS, step, (lo0, hi0))
    thr = lax.bitcast_convert_type(lo, jnp.float32)
    thr_ref[...] = jnp.broadcast_to(thr, (_B, _L))


def _prep_call(scores, *, interpret=False):
    return pl.pallas_call(
        _prep_body,
        out_shape=jax.ShapeDtypeStruct((_B, _L), jnp.float32),
        interpret=interpret,
    )(scores)


def _make_compact_gather(interpret=False):
    @functools.partial(
        pl.kernel,
        out_type=(jax.ShapeDtypeStruct((_B * _K,), jnp.int32),
                  jax.ShapeDtypeStruct((_B * _K, _C), jnp.float32)),
        mesh=_sc_mesh(),
        scratch_types=[
            pltpu.VMEM((_N,), jnp.float32),
            pltpu.VMEM((_L,), jnp.float32),
            pltpu.VMEM((2 * _K + _L,), jnp.int32),
            pltpu.VMEM((_K + _L,), jnp.int32),
            pltpu.VMEM((_RPW,), jnp.int32),
            pltpu.VMEM((_RPW, _C), jnp.float32),
            pltpu.VMEM_SHARED((2 * _K,), jnp.int32),
            pltpu.SemaphoreType.DMA,
        ],
        compiler_params=_SC_PARAMS,
        interpret=interpret,
    )
    def compact_gather(scores_hbm, thr_hbm, xflat_hbm, fidx_hbm, xf_hbm,
                       sc_v, thr_v, idxbuf, eqbuf, idx_v, rows_v,
                       shared_idx, sem):
        cid = lax.axis_index("c")
        sid = lax.axis_index("s")
        w = cid * _NS + sid

        @pl.when(jnp.logical_or(sid == 0, sid == 8))
        def _():
            b = cid * 2 + sid // 8
            pltpu.sync_copy(scores_hbm.at[b], sc_v)
            pltpu.sync_copy(thr_hbm.at[b], thr_v)
            thr_vec = thr_v[...]
            iota = lax.iota(jnp.int32, _L)
            base = b * _N

            trash_gt = 2 * _K + _L - 1
            trash_eq = _K

            def step(i, carry):
                off_gt, off_eq = carry
                v = sc_v[pl.ds(i * _L, _L)]
                idxv = iota + (base + i * _L)
                mgt = v > thr_vec
                inc_gt = mgt.astype(jnp.int32)
                cs_gt = plsc.cumsum(inc_gt)
                pos_gt = jnp.where(mgt, off_gt + cs_gt - 1, trash_gt)
                plsc.store_scatter(idxbuf, [pos_gt], idxv)
                ngt = jnp.sum(inc_gt, axis=0)
                meq = v == thr_vec
                inc_eq = meq.astype(jnp.int32)
                cs_eq = plsc.cumsum(inc_eq)
                pos_raw = off_eq + cs_eq - 1
                valid = meq & (pos_raw < _K)
                pos_eq = jnp.where(valid, pos_raw, trash_eq)
                plsc.store_scatter(eqbuf, [pos_eq], idxv)
                neq = jnp.sum(inc_eq, axis=0)
                return off_gt + ngt, off_eq + neq

            off_gt, _unused = lax.fori_loop(
                0, _N // _L, step, (jnp.int32(0), jnp.int32(0)))

            def app(j, carry):
                idxbuf[pl.ds(off_gt + j * _L, _L)] = eqbuf[pl.ds(j * _L, _L)]
                return carry

            lax.fori_loop(0, _K // _L, app, jnp.int32(0))
            pltpu.sync_copy(idxbuf.at[pl.ds(0, _K)],
                            shared_idx.at[pl.ds((sid // 8) * _K, _K)])

        plsc.subcore_barrier()

        pltpu.sync_copy(shared_idx.at[pl.ds(sid * _RPW, _RPW)], idx_v)
        pltpu.async_copy(xflat_hbm.at[idx_v], rows_v, sem).wait()
        base_out = w * _RPW
        pltpu.sync_copy(rows_v, xf_hbm.at[pl.ds(base_out, _RPW)])

        @pl.when(jnp.logical_or(sid == 0, sid == 8))
        def _():
            b = cid * 2 + sid // 8
            pltpu.sync_copy(idxbuf.at[pl.ds(0, _K)],
                            fidx_hbm.at[pl.ds(b * _K, _K)])

    return compact_gather


_GF = 8
_RB = 2048
_GC = 8
_TM = (_B * _K) // _GF
_NROWS = _B * _N
_RA = _NROWS - _GC * _RB
_RAB = 2048
_OFF = _RA // _RB


def _copy_a_body(src_ref, dst_ref):
    dst_ref[...] = src_ref[...]


def _copy_a_call(cache_flat, *, interpret=False):
    return pl.pallas_call(
        _copy_a_body,
        grid=(_RA // _RAB,),
        in_specs=[pl.BlockSpec((_RAB, _C), lambda i: (i, 0))],
        out_specs=pl.BlockSpec((_RAB, _C), lambda i: (i, 0)),
        out_shape=jax.ShapeDtypeStruct((_NROWS, _C), jnp.float32),
        interpret=interpret,
    )(cache_flat)


def _fused_body(alias_ref, cache_ref, xf_ref, w1_ref, b1_ref, w2_ref,
                b2_ref, copy_ref, out_ref):
    del alias_ref

    @pl.when(pl.program_id(0) < _GC)
    def _():
        copy_ref[...] = cache_ref[...]
    h = jnp.dot(xf_ref[...].astype(jnp.bfloat16),
                w1_ref[...].astype(jnp.bfloat16),
                preferred_element_type=jnp.float32)
    g = jax.nn.gelu((h + b1_ref[...]).astype(jnp.bfloat16))
    out = jnp.dot(g, w2_ref[...].astype(jnp.bfloat16),
                  preferred_element_type=jnp.float32)
    out_ref[...] = out + b2_ref[...]


def _fused_call(copied_a, cache_flat, xf, w1, b1, w2, b2, *, interpret=False):
    return pl.pallas_call(
        _fused_body,
        grid=(_GF,),
        in_specs=[
            pl.BlockSpec(memory_space=pl.ANY),
            pl.BlockSpec((_RB, _C),
                         lambda i: (jnp.minimum(i, _GC - 1) + _OFF, 0)),
            pl.BlockSpec((_TM, _C), lambda i: (i, 0)),
            pl.BlockSpec((_C, _DFF), lambda i: (0, 0)),
            pl.BlockSpec((1, _DFF), lambda i: (0, 0)),
            pl.BlockSpec((_DFF, _C), lambda i: (0, 0)),
            pl.BlockSpec((1, _C), lambda i: (0, 0)),
        ],
        out_specs=[
            pl.BlockSpec((_RB, _C),
                         lambda i: (jnp.minimum(i, _GC - 1) + _OFF, 0)),
            pl.BlockSpec((_TM, _C), lambda i: (i, 0)),
        ],
        out_shape=[
            jax.ShapeDtypeStruct((_NROWS, _C), jnp.float32),
            jax.ShapeDtypeStruct((_B * _K, _C), jnp.float32),
        ],
        input_output_aliases={0: 0},
        compiler_params=pltpu.CompilerParams(
            vmem_limit_bytes=100 * 1024 * 1024),
        interpret=interpret,
    )(copied_a, cache_flat, xf, w1, b1, w2, b2)


def _make_scatter(interpret=False):
    @functools.partial(
        pl.kernel,
        out_type=(),
        mesh=_sc_mesh(),
        scratch_types=[
            pltpu.VMEM((_RPW,), jnp.int32),
            pltpu.VMEM((_RPW, _C), jnp.float32),
            pltpu.SemaphoreType.DMA,
        ],
        compiler_params=_SC_PARAMS,
        interpret=interpret,
    )
    def scatter(fidx_hbm, outf_hbm, dst_ref, idx_v, rows_v, sem):
        base = _wid() * _RPW
        pltpu.sync_copy(fidx_hbm.at[pl.ds(base, _RPW)], idx_v)
        pltpu.sync_copy(outf_hbm.at[pl.ds(base, _RPW)], rows_v)
        pltpu.async_copy(rows_v, dst_ref.at[idx_v], sem).wait()

    return scatter


def kernel(x, cache, scores, W1, b1, W2, b2):
    xflat = x.reshape(_B * _N, _C)
    cache_flat = cache.reshape(_NROWS, _C)
    copied_a = _copy_a_call(cache_flat)
    thr = _prep_call(scores)
    fidx, xf = _make_compact_gather()(scores, thr, xflat)
    copied, outf = _fused_call(copied_a, cache_flat, xf, W1,
                               b1.reshape(1, _DFF), W2, b2.reshape(1, _C))
    dst = jax.new_ref(copied)
    _make_scatter()(fidx, outf, dst)
    return dst[...].reshape(_B, _N, _C)


# speedup vs baseline: 1.0953x; 1.0022x over previous
import functools

import jax
import jax.numpy as jnp
from jax import lax
from jax.experimental import pallas as pl
from jax.experimental.pallas import tpu as pltpu
from jax.experimental.pallas import tpu_sc as plsc

_B, _N, _C, _DFF, _K = 4, 8192, 768, 3072, 1024
_NC, _NS, _L = 2, 16, 16
_NW = _NC * _NS
_RPW = (_B * _K) // _NW
_SEARCH_ITERS = 31


def _sc_mesh():
    return plsc.VectorSubcoreMesh(
        core_axis_name="c", subcore_axis_name="s",
        num_cores=_NC, num_subcores=_NS)


_SC_PARAMS = pltpu.CompilerParams(needs_layout_passes=False)


def _wid():
    return lax.axis_index("s") * _NC + lax.axis_index("c")


def _prep_body(scores_ref, thr_ref):
    bits = lax.bitcast_convert_type(scores_ref[...], jnp.int32)

    def step(_, carry):
        lo, hi = carry
        mid = lo + (hi - lo) // 2
        cnt = jnp.sum((bits >= mid).astype(jnp.int32), axis=1, keepdims=True)
        ok = cnt >= _K
        return jnp.where(ok, mid, lo), jnp.where(ok, hi, mid)

    lo0 = jnp.zeros((_B, 1), jnp.int32)
    hi0 = jnp.full((_B, 1), 0x7F800000, jnp.int32)
    lo, _ = lax.fori_loop(0, _SEARCH_ITERS, step, (lo0, hi0))
    thr = lax.bitcast_convert_type(lo, jnp.float32)
    thr_ref[...] = jnp.broadcast_to(thr, (_B, _L))


def _prep_call(scores, *, interpret=False):
    return pl.pallas_call(
        _prep_body,
        out_shape=jax.ShapeDtypeStruct((_B, _L), jnp.float32),
        interpret=interpret,
    )(scores)


def _make_compact_gather(interpret=False):
    @functools.partial(
        pl.kernel,
        out_type=(jax.ShapeDtypeStruct((_B * _K,), jnp.int32),
                  jax.ShapeDtypeStruct((_B * _K, _C), jnp.float32)),
        mesh=_sc_mesh(),
        scratch_types=[
            pltpu.VMEM((_N,), jnp.float32),
            pltpu.VMEM((_L,), jnp.float32),
            pltpu.VMEM((2 * _K + _L,), jnp.int32),
            pltpu.VMEM((_K + _L,), jnp.int32),
            pltpu.VMEM((_RPW,), jnp.int32),
            pltpu.VMEM((_RPW, _C), jnp.float32),
            pltpu.VMEM_SHARED((2 * _K,), jnp.int32),
            pltpu.SemaphoreType.DMA,
        ],
        compiler_params=_SC_PARAMS,
        interpret=interpret,
    )
    def compact_gather(scores_hbm, thr_hbm, xflat_hbm, fidx_hbm, xf_hbm,
                       sc_v, thr_v, idxbuf, eqbuf, idx_v, rows_v,
                       shared_idx, sem):
        cid = lax.axis_index("c")
        sid = lax.axis_index("s")
        w = cid * _NS + sid

        @pl.when(jnp.logical_or(sid == 0, sid == 8))
        def _():
            b = cid * 2 + sid // 8
            pltpu.sync_copy(scores_hbm.at[b], sc_v)
            pltpu.sync_copy(thr_hbm.at[b], thr_v)
            thr_vec = thr_v[...]
            iota = lax.iota(jnp.int32, _L)
            base = b * _N

            trash_gt = 2 * _K + _L - 1
            trash_eq = _K

            def step(i, carry):
                off_gt, off_eq = carry
                v = sc_v[pl.ds(i * _L, _L)]
                idxv = iota + (base + i * _L)
                mgt = v > thr_vec
                inc_gt = mgt.astype(jnp.int32)
                cs_gt = plsc.cumsum(inc_gt)
                pos_gt = jnp.where(mgt, off_gt + cs_gt - 1, trash_gt)
                plsc.store_scatter(idxbuf, [pos_gt], idxv)
                ngt = jnp.sum(inc_gt, axis=0)
                meq = v == thr_vec
                inc_eq = meq.astype(jnp.int32)
                cs_eq = plsc.cumsum(inc_eq)
                pos_raw = off_eq + cs_eq - 1
                valid = meq & (pos_raw < _K)
                pos_eq = jnp.where(valid, pos_raw, trash_eq)
                plsc.store_scatter(eqbuf, [pos_eq], idxv)
                neq = jnp.sum(inc_eq, axis=0)
                return off_gt + ngt, off_eq + neq

            off_gt, _unused = lax.fori_loop(
                0, _N // _L, step, (jnp.int32(0), jnp.int32(0)))

            def app(j, carry):
                idxbuf[pl.ds(off_gt + j * _L, _L)] = eqbuf[pl.ds(j * _L, _L)]
                return carry

            lax.fori_loop(0, _K // _L, app, jnp.int32(0))
            pltpu.sync_copy(idxbuf.at[pl.ds(0, _K)],
                            shared_idx.at[pl.ds((sid // 8) * _K, _K)])

        plsc.subcore_barrier()

        pltpu.sync_copy(shared_idx.at[pl.ds(sid * _RPW, _RPW)], idx_v)
        pltpu.async_copy(xflat_hbm.at[idx_v], rows_v, sem).wait()
        base_out = w * _RPW
        pltpu.sync_copy(rows_v, xf_hbm.at[pl.ds(base_out, _RPW)])

        @pl.when(jnp.logical_or(sid == 0, sid == 8))
        def _():
            b = cid * 2 + sid // 8
            pltpu.sync_copy(idxbuf.at[pl.ds(0, _K)],
                            fidx_hbm.at[pl.ds(b * _K, _K)])

    return compact_gather


_GF = 8
_RB = 2048
_GC = 8
_TM = (_B * _K) // _GF
_NROWS = _B * _N
_RA = _NROWS - _GC * _RB
_RAB = 4096
_OFF = _RA // _RB


def _copy_a_body(src_ref, dst_ref):
    dst_ref[...] = src_ref[...]


def _copy_a_call(cache_flat, *, interpret=False):
    return pl.pallas_call(
        _copy_a_body,
        grid=(_RA // _RAB,),
        in_specs=[pl.BlockSpec((_RAB, _C), lambda i: (i, 0))],
        out_specs=pl.BlockSpec((_RAB, _C), lambda i: (i, 0)),
        out_shape=jax.ShapeDtypeStruct((_NROWS, _C), jnp.float32),
        compiler_params=pltpu.CompilerParams(
            vmem_limit_bytes=100 * 1024 * 1024),
        interpret=interpret,
    )(cache_flat)


def _fused_body(alias_ref, cache_ref, xf_ref, w1_ref, b1_ref, w2_ref,
                b2_ref, copy_ref, out_ref):
    del alias_ref

    @pl.when(pl.program_id(0) < _GC)
    def _():
        copy_ref[...] = cache_ref[...]
    h = jnp.dot(xf_ref[...].astype(jnp.bfloat16),
                w1_ref[...].astype(jnp.bfloat16),
                preferred_element_type=jnp.float32)
    g = jax.nn.gelu((h + b1_ref[...]).astype(jnp.bfloat16))
    out = jnp.dot(g, w2_ref[...].astype(jnp.bfloat16),
                  preferred_element_type=jnp.float32)
    out_ref[...] = out + b2_ref[...]


def _fused_call(copied_a, cache_flat, xf, w1, b1, w2, b2, *, interpret=False):
    return pl.pallas_call(
        _fused_body,
        grid=(_GF,),
        in_specs=[
            pl.BlockSpec(memory_space=pl.ANY),
            pl.BlockSpec((_RB, _C),
                         lambda i: (jnp.minimum(i, _GC - 1) + _OFF, 0)),
            pl.BlockSpec((_TM, _C), lambda i: (i, 0)),
            pl.BlockSpec((_C, _DFF), lambda i: (0, 0)),
            pl.BlockSpec((1, _DFF), lambda i: (0, 0)),
            pl.BlockSpec((_DFF, _C), lambda i: (0, 0)),
            pl.BlockSpec((1, _C), lambda i: (0, 0)),
        ],
        out_specs=[
            pl.BlockSpec((_RB, _C),
                         lambda i: (jnp.minimum(i, _GC - 1) + _OFF, 0)),
            pl.BlockSpec((_TM, _C), lambda i: (i, 0)),
        ],
        out_shape=[
            jax.ShapeDtypeStruct((_NROWS, _C), jnp.float32),
            jax.ShapeDtypeStruct((_B * _K, _C), jnp.float32),
        ],
        input_output_aliases={0: 0},
        compiler_params=pltpu.CompilerParams(
            vmem_limit_bytes=100 * 1024 * 1024),
        interpret=interpret,
    )(copied_a, cache_flat, xf, w1, b1, w2, b2)


def _make_scatter(interpret=False):
    @functools.partial(
        pl.kernel,
        out_type=(),
        mesh=_sc_mesh(),
        scratch_types=[
            pltpu.VMEM((_RPW,), jnp.int32),
            pltpu.VMEM((_RPW, _C), jnp.float32),
            pltpu.SemaphoreType.DMA,
        ],
        compiler_params=_SC_PARAMS,
        interpret=interpret,
    )
    def scatter(fidx_hbm, outf_hbm, dst_ref, idx_v, rows_v, sem):
        base = _wid() * _RPW
        pltpu.sync_copy(fidx_hbm.at[pl.ds(base, _RPW)], idx_v)
        pltpu.sync_copy(outf_hbm.at[pl.ds(base, _RPW)], rows_v)
        pltpu.async_copy(rows_v, dst_ref.at[idx_v], sem).wait()

    return scatter


def kernel(x, cache, scores, W1, b1, W2, b2):
    xflat = x.reshape(_B * _N, _C)
    cache_flat = cache.reshape(_NROWS, _C)
    copied_a = _copy_a_call(cache_flat)
    thr = _prep_call(scores)
    fidx, xf = _make_compact_gather()(scores, thr, xflat)
    copied, outf = _fused_call(copied_a, cache_flat, xf, W1,
                               b1.reshape(1, _DFF), W2, b2.reshape(1, _C))
    dst = jax.new_ref(copied)
    _make_scatter()(fidx, outf, dst)
    return dst[...].reshape(_B, _N, _C)
